# Initial kernel scaffold; baseline (speedup 1.0000x reference)
import jax
import jax.numpy as jnp
from jax.experimental import pallas as pl


def kernel(accuracies, confidences):
    raise NotImplementedError("write your pallas kernel here")


# speedup vs baseline: 6.5609x; 6.5609x over previous
import functools

import jax
import jax.numpy as jnp
from jax import lax
from jax.experimental import pallas as pl
from jax.experimental.pallas import tpu as pltpu
from jax.experimental.pallas import tpu_sc as plsc

N_BINS = 64

_info = plsc.get_sparse_core_info()
_NC, _NS, _L = _info.num_cores, _info.num_subcores, _info.num_lanes
_NW = _NC * _NS


def _sc_body(acc_hbm, conf_hbm, cnt_out, sd_out, acc_v, conf_v, cnt_ref,
             sd_ref, n_per_w, chunk):
    wid = lax.axis_index("s") * _NC + lax.axis_index("c")
    base = wid * n_per_w

    zeros16 = jnp.zeros((_L,), jnp.float32)
    for b in range(N_BINS):
        cnt_ref[b, :] = zeros16
        sd_ref[b, :] = zeros16

    lane = lax.iota(jnp.int32, _L)
    ones16 = jnp.ones((_L,), jnp.float32)
    n_chunks = n_per_w // chunk

    def do_chunk(c, _):
        off = base + c * chunk
        pltpu.sync_copy(acc_hbm.at[pl.ds(off, chunk)], acc_v)
        pltpu.sync_copy(conf_hbm.at[pl.ds(off, chunk)], conf_v)

        def inner(i, _):
            conf = conf_v[pl.ds(i * _L, _L)]
            acc = acc_v[pl.ds(i * _L, _L)]
            bin_ = jnp.minimum((conf * jnp.float32(N_BINS)).astype(jnp.int32),
                               N_BINS - 1)
            plsc.addupdate_scatter(cnt_ref, [bin_, lane], ones16)
            plsc.addupdate_scatter(sd_ref, [bin_, lane], conf - acc)
            return 0

        lax.fori_loop(0, chunk // _L, inner, 0)
        return 0

    lax.fori_loop(0, n_chunks, do_chunk, 0)

    pltpu.sync_copy(cnt_ref, cnt_out.at[wid])
    pltpu.sync_copy(sd_ref, sd_out.at[wid])


def _final_body(cnt_ref, sd_ref, o_ref, *, n_total):
    cnt = jnp.sum(cnt_ref[...], axis=(0, 2))
    sd = jnp.sum(sd_ref[...], axis=(0, 2))
    denom = jnp.maximum(cnt, 1.0)
    contrib = jnp.where(cnt > 0.0,
                        (cnt / jnp.float32(n_total)) * jnp.abs(sd) / denom,
                        0.0)
    o_ref[...] = jnp.full((8, 128), jnp.sum(contrib), jnp.float32)


def kernel(accuracies, confidences):
    n = confidences.shape[0]
    n_per_w = n // _NW
    chunk = 16384
    if n_per_w % chunk != 0:
        chunk = n_per_w

    mesh = plsc.VectorSubcoreMesh(core_axis_name="c", subcore_axis_name="s")
    sc_fn = pl.kernel(
        functools.partial(_sc_body, n_per_w=n_per_w, chunk=chunk),
        mesh=mesh,
        compiler_params=pltpu.CompilerParams(needs_layout_passes=False),
        out_type=(
            jax.ShapeDtypeStruct((_NW, N_BINS, _L), jnp.float32),
            jax.ShapeDtypeStruct((_NW, N_BINS, _L), jnp.float32),
        ),
        scratch_types=[
            pltpu.VMEM((chunk,), jnp.float32),
            pltpu.VMEM((chunk,), jnp.float32),
            pltpu.VMEM((N_BINS, _L), jnp.float32),
            pltpu.VMEM((N_BINS, _L), jnp.float32),
        ],
    )
    cnt_parts, sd_parts = sc_fn(accuracies, confidences)

    out = pl.pallas_call(
        functools.partial(_final_body, n_total=n),
        out_shape=jax.ShapeDtypeStruct((8, 128), jnp.float32),
    )(cnt_parts, sd_parts)
    return out[0, :1]


# speedup vs baseline: 7.0437x; 1.0736x over previous
import functools

import jax
import jax.numpy as jnp
from jax import lax
from jax.experimental import pallas as pl
from jax.experimental.pallas import tpu as pltpu
from jax.experimental.pallas import tpu_sc as plsc

N_BINS = 64

_info = plsc.get_sparse_core_info()
_NC, _NS, _L = _info.num_cores, _info.num_subcores, _info.num_lanes
_NW = _NC * _NS


def _sc_body(acc_hbm, conf_hbm, cnt_out, sd_out, acc_v0, conf_v0, acc_v1,
             conf_v1, cnt_ref, sd_ref, sem_a0, sem_c0, sem_a1, sem_c1,
             n_per_w, chunk, unroll):
    wid = lax.axis_index("s") * _NC + lax.axis_index("c")
    base = wid * n_per_w

    zeros16 = jnp.zeros((_L,), jnp.float32)
    for b in range(N_BINS):
        cnt_ref[b, :] = zeros16
        sd_ref[b, :] = zeros16

    lane = lax.iota(jnp.int32, _L)
    ones16 = jnp.ones((_L,), jnp.float32)
    n_chunks = n_per_w // chunk
    bufs = [(acc_v0, conf_v0, sem_a0, sem_c0),
            (acc_v1, conf_v1, sem_a1, sem_c1)]

    def start(c):
        av, cv, sa, sc = bufs[c % 2]
        off = base + c * chunk
        return (pltpu.async_copy(acc_hbm.at[pl.ds(off, chunk)], av, sa),
                pltpu.async_copy(conf_hbm.at[pl.ds(off, chunk)], cv, sc))

    group = _L * unroll
    handles = start(0)
    for c in range(n_chunks):
        nxt = start(c + 1) if c + 1 < n_chunks else None
        handles[0].wait()
        handles[1].wait()
        av, cv = bufs[c % 2][0], bufs[c % 2][1]

        def inner(i, _, av=av, cv=cv):
            o0 = i * group
            for k in range(unroll):
                conf = cv[pl.ds(o0 + k * _L, _L)]
                acc = av[pl.ds(o0 + k * _L, _L)]
                bin_ = jnp.minimum(
                    (conf * jnp.float32(N_BINS)).astype(jnp.int32), N_BINS - 1)
                plsc.addupdate_scatter(cnt_ref, [bin_, lane], ones16)
                plsc.addupdate_scatter(sd_ref, [bin_, lane], conf - acc)
            return 0

        lax.fori_loop(0, chunk // group, inner, 0)
        handles = nxt

    pltpu.sync_copy(cnt_ref, cnt_out.at[wid])
    pltpu.sync_copy(sd_ref, sd_out.at[wid])


def _final_body(cnt_ref, sd_ref, o_ref, *, n_total):
    cnt = jnp.sum(cnt_ref[...], axis=(0, 2))
    sd = jnp.sum(sd_ref[...], axis=(0, 2))
    denom = jnp.maximum(cnt, 1.0)
    contrib = jnp.where(cnt > 0.0,
                        (cnt / jnp.float32(n_total)) * jnp.abs(sd) / denom,
                        0.0)
    o_ref[...] = jnp.full((8, 128), jnp.sum(contrib), jnp.float32)


def kernel(accuracies, confidences):
    n = confidences.shape[0]
    n_per_w = n // _NW
    chunk = 16384
    if n_per_w % chunk != 0:
        chunk = n_per_w

    mesh = plsc.VectorSubcoreMesh(core_axis_name="c", subcore_axis_name="s")
    sc_fn = pl.kernel(
        functools.partial(_sc_body, n_per_w=n_per_w, chunk=chunk, unroll=8),
        mesh=mesh,
        compiler_params=pltpu.CompilerParams(needs_layout_passes=False),
        out_type=(
            jax.ShapeDtypeStruct((_NW, N_BINS, _L), jnp.float32),
            jax.ShapeDtypeStruct((_NW, N_BINS, _L), jnp.float32),
        ),
        scratch_types=[
            pltpu.VMEM((chunk,), jnp.float32),
            pltpu.VMEM((chunk,), jnp.float32),
            pltpu.VMEM((chunk,), jnp.float32),
            pltpu.VMEM((chunk,), jnp.float32),
            pltpu.VMEM((N_BINS, _L), jnp.float32),
            pltpu.VMEM((N_BINS, _L), jnp.float32),
            pltpu.SemaphoreType.DMA,
            pltpu.SemaphoreType.DMA,
            pltpu.SemaphoreType.DMA,
            pltpu.SemaphoreType.DMA,
        ],
    )
    cnt_parts, sd_parts = sc_fn(accuracies, confidences)

    out = pl.pallas_call(
        functools.partial(_final_body, n_total=n),
        out_shape=jax.ShapeDtypeStruct((8, 128), jnp.float32),
    )(cnt_parts, sd_parts)
    return out[0, :1]


# speedup vs baseline: 13.5950x; 1.9301x over previous
import functools

import jax
import jax.numpy as jnp
from jax import lax
from jax.experimental import pallas as pl
from jax.experimental.pallas import tpu as pltpu
from jax.experimental.pallas import tpu_sc as plsc

N_BINS = 64

_info = plsc.get_sparse_core_info()
_NC, _NS, _L = _info.num_cores, _info.num_subcores, _info.num_lanes
_NW = _NC * _NS


def _sc_body(acc_hbm, conf_hbm, cnt_out, sd_out, acc_v0, conf_v0, acc_v1,
             conf_v1, cnt_ref, sd_ref, sem_a0, sem_c0, sem_a1, sem_c1,
             n_per_w, chunk, unroll):
    wid = lax.axis_index("s") * _NC + lax.axis_index("c")
    base = wid * n_per_w

    zeros16 = jnp.zeros((_L,), jnp.float32)
    for b in range(N_BINS):
        cnt_ref[b, :] = zeros16
        sd_ref[b, :] = zeros16

    lane = lax.iota(jnp.int32, _L)
    ones16 = jnp.ones((_L,), jnp.float32)
    n_chunks = n_per_w // chunk
    bufs = [(acc_v0, conf_v0, sem_a0, sem_c0),
            (acc_v1, conf_v1, sem_a1, sem_c1)]

    def start(c):
        av, cv, sa, sc = bufs[c % 2]
        off = base + c * chunk
        return (pltpu.async_copy(acc_hbm.at[pl.ds(off, chunk)], av, sa),
                pltpu.async_copy(conf_hbm.at[pl.ds(off, chunk)], cv, sc))

    group = _L * unroll
    handles = start(0)
    for c in range(n_chunks):
        nxt = start(c + 1) if c + 1 < n_chunks else None
        handles[0].wait()
        handles[1].wait()
        av, cv = bufs[c % 2][0], bufs[c % 2][1]

        def inner(i, _, av=av, cv=cv):
            o0 = i * group
            confs = [cv[pl.ds(o0 + k * _L, _L)] for k in range(unroll)]
            accs = [av[pl.ds(o0 + k * _L, _L)] for k in range(unroll)]
            bins = [jnp.minimum(
                (c * jnp.float32(N_BINS)).astype(jnp.int32), N_BINS - 1)
                for c in confs]
            diffs = [c - a for c, a in zip(confs, accs)]
            for k in range(unroll):
                plsc.addupdate_scatter(cnt_ref, [bins[k], lane], ones16)
                plsc.addupdate_scatter(sd_ref, [bins[k], lane], diffs[k])
            return 0

        lax.fori_loop(0, chunk // group, inner, 0)
        handles = nxt

    pltpu.sync_copy(cnt_ref, cnt_out.at[wid])
    pltpu.sync_copy(sd_ref, sd_out.at[wid])


def _final_body(cnt_ref, sd_ref, o_ref, *, n_total):
    cnt = jnp.sum(cnt_ref[...], axis=(0, 2))
    sd = jnp.sum(sd_ref[...], axis=(0, 2))
    denom = jnp.maximum(cnt, 1.0)
    contrib = jnp.where(cnt > 0.0,
                        (cnt / jnp.float32(n_total)) * jnp.abs(sd) / denom,
                        0.0)
    o_ref[...] = jnp.full((8, 128), jnp.sum(contrib), jnp.float32)


def kernel(accuracies, confidences):
    n = confidences.shape[0]
    n_per_w = n // _NW
    chunk = 16384
    if n_per_w % chunk != 0:
        chunk = n_per_w

    mesh = plsc.VectorSubcoreMesh(core_axis_name="c", subcore_axis_name="s")
    sc_fn = pl.kernel(
        functools.partial(_sc_body, n_per_w=n_per_w, chunk=chunk, unroll=8),
        mesh=mesh,
        compiler_params=pltpu.CompilerParams(needs_layout_passes=False),
        out_type=(
            jax.ShapeDtypeStruct((_NW, N_BINS, _L), jnp.float32),
            jax.ShapeDtypeStruct((_NW, N_BINS, _L), jnp.float32),
        ),
        scratch_types=[
            pltpu.VMEM((chunk,), jnp.float32),
            pltpu.VMEM((chunk,), jnp.float32),
            pltpu.VMEM((chunk,), jnp.float32),
            pltpu.VMEM((chunk,), jnp.float32),
            pltpu.VMEM((N_BINS, _L), jnp.float32),
            pltpu.VMEM((N_BINS, _L), jnp.float32),
            pltpu.SemaphoreType.DMA,
            pltpu.SemaphoreType.DMA,
            pltpu.SemaphoreType.DMA,
            pltpu.SemaphoreType.DMA,
        ],
    )
    cnt_parts, sd_parts = sc_fn(accuracies, confidences)

    out = pl.pallas_call(
        functools.partial(_final_body, n_total=n),
        out_shape=jax.ShapeDtypeStruct((8, 128), jnp.float32),
    )(cnt_parts, sd_parts)
    return out[0, :1]


# speedup vs baseline: 14.2920x; 1.0513x over previous
import functools

import jax
import jax.numpy as jnp
from jax import lax
from jax.experimental import pallas as pl
from jax.experimental.pallas import tpu as pltpu
from jax.experimental.pallas import tpu_sc as plsc

N_BINS = 64

_info = plsc.get_sparse_core_info()
_NC, _NS, _L = _info.num_cores, _info.num_subcores, _info.num_lanes
_NW = _NC * _NS


def _sc_body(acc_hbm, conf_hbm, cnt_out, sd_out, acc_v0, conf_v0, acc_v1,
             conf_v1, cnt_ref, sd_ref, sem_a0, sem_c0, sem_a1, sem_c1,
             n_per_w, chunk, unroll):
    wid = lax.axis_index("s") * _NC + lax.axis_index("c")
    base = wid * n_per_w

    zeros16 = jnp.zeros((_L,), jnp.float32)
    for b in range(N_BINS):
        cnt_ref[b, :] = zeros16
        sd_ref[b, :] = zeros16

    lane = lax.iota(jnp.int32, _L)
    ones16 = jnp.ones((_L,), jnp.float32)
    n_chunks = n_per_w // chunk
    bufs = [(acc_v0, conf_v0, sem_a0, sem_c0),
            (acc_v1, conf_v1, sem_a1, sem_c1)]

    def start(c):
        av, cv, sa, sc = bufs[c % 2]
        off = base + c * chunk
        return (pltpu.async_copy(acc_hbm.at[pl.ds(off, chunk)], av, sa),
                pltpu.async_copy(conf_hbm.at[pl.ds(off, chunk)], cv, sc))

    group = _L * unroll
    handles = start(0)
    for c in range(n_chunks):
        nxt = start(c + 1) if c + 1 < n_chunks else None
        handles[0].wait()
        handles[1].wait()
        av, cv = bufs[c % 2][0], bufs[c % 2][1]

        def inner(i, _, av=av, cv=cv):
            o0 = i * group
            confs = [cv[pl.ds(o0 + k * _L, _L)] for k in range(unroll)]
            accs = [av[pl.ds(o0 + k * _L, _L)] for k in range(unroll)]
            bins = [jnp.minimum(
                (c * jnp.float32(N_BINS)).astype(jnp.int32), N_BINS - 1)
                for c in confs]
            diffs = [c - a for c, a in zip(confs, accs)]
            for k in range(unroll):
                plsc.addupdate_scatter(cnt_ref, [bins[k], lane], ones16)
                plsc.addupdate_scatter(sd_ref, [bins[k], lane], diffs[k])
            return 0

        lax.fori_loop(0, chunk // group, inner, 0)
        handles = nxt

    pltpu.sync_copy(cnt_ref, cnt_out.at[wid])
    pltpu.sync_copy(sd_ref, sd_out.at[wid])


def _final_body(cnt_ref, sd_ref, o_ref, *, n_total):
    cnt = jnp.sum(cnt_ref[...], axis=(0, 2))
    sd = jnp.sum(sd_ref[...], axis=(0, 2))
    denom = jnp.maximum(cnt, 1.0)
    contrib = jnp.where(cnt > 0.0,
                        (cnt / jnp.float32(n_total)) * jnp.abs(sd) / denom,
                        0.0)
    o_ref[0] = jnp.sum(contrib)


def kernel(accuracies, confidences):
    n = confidences.shape[0]
    n_per_w = n // _NW
    chunk = 8192
    if n_per_w % chunk != 0:
        chunk = n_per_w

    mesh = plsc.VectorSubcoreMesh(core_axis_name="c", subcore_axis_name="s")
    sc_fn = pl.kernel(
        functools.partial(_sc_body, n_per_w=n_per_w, chunk=chunk, unroll=16),
        mesh=mesh,
        compiler_params=pltpu.CompilerParams(needs_layout_passes=False),
        out_type=(
            jax.ShapeDtypeStruct((_NW, N_BINS, _L), jnp.float32),
            jax.ShapeDtypeStruct((_NW, N_BINS, _L), jnp.float32),
        ),
        scratch_types=[
            pltpu.VMEM((chunk,), jnp.float32),
            pltpu.VMEM((chunk,), jnp.float32),
            pltpu.VMEM((chunk,), jnp.float32),
            pltpu.VMEM((chunk,), jnp.float32),
            pltpu.VMEM((N_BINS, _L), jnp.float32),
            pltpu.VMEM((N_BINS, _L), jnp.float32),
            pltpu.SemaphoreType.DMA,
            pltpu.SemaphoreType.DMA,
            pltpu.SemaphoreType.DMA,
            pltpu.SemaphoreType.DMA,
        ],
    )
    cnt_parts, sd_parts = sc_fn(accuracies, confidences)

    out = pl.pallas_call(
        functools.partial(_final_body, n_total=n),
        out_shape=jax.ShapeDtypeStruct((1,), jnp.float32),
        out_specs=pl.BlockSpec(memory_space=pltpu.SMEM),
    )(cnt_parts, sd_parts)
    return out


# speedup vs baseline: 14.2960x; 1.0003x over previous
import functools

import jax
import jax.numpy as jnp
from jax import lax
from jax.experimental import pallas as pl
from jax.experimental.pallas import tpu as pltpu
from jax.experimental.pallas import tpu_sc as plsc

N_BINS = 64

_info = plsc.get_sparse_core_info()
_NC, _NS, _L = _info.num_cores, _info.num_subcores, _info.num_lanes
_NW = _NC * _NS


def _sc_body(acc_hbm, conf_hbm, cnt_out, sd_out, bufs_and_sems, cnt_ref,
             sd_ref, n_per_w, chunk, unroll, nbuf):
    wid = lax.axis_index("s") * _NC + lax.axis_index("c")
    base = wid * n_per_w

    zeros16 = jnp.zeros((_L,), jnp.float32)
    for b in range(N_BINS):
        cnt_ref[b, :] = zeros16
        sd_ref[b, :] = zeros16

    lane = lax.iota(jnp.int32, _L)
    ones16 = jnp.ones((_L,), jnp.float32)
    n_chunks = n_per_w // chunk
    bufs = [tuple(bufs_and_sems[4 * i:4 * i + 4]) for i in range(nbuf)]

    def start(c):
        av, cv, sa, sc = bufs[c % nbuf]
        off = base + c * chunk
        return (pltpu.async_copy(acc_hbm.at[pl.ds(off, chunk)], av, sa),
                pltpu.async_copy(conf_hbm.at[pl.ds(off, chunk)], cv, sc))

    group = _L * unroll
    pending = [start(c) for c in range(min(nbuf - 1, n_chunks))]
    for c in range(n_chunks):
        if c + nbuf - 1 < n_chunks:
            pending.append(start(c + nbuf - 1))
        handles = pending.pop(0)
        handles[0].wait()
        handles[1].wait()
        av, cv = bufs[c % nbuf][0], bufs[c % nbuf][1]

        def inner(i, _, av=av, cv=cv):
            o0 = i * group
            confs = [cv[pl.ds(o0 + k * _L, _L)] for k in range(unroll)]
            accs = [av[pl.ds(o0 + k * _L, _L)] for k in range(unroll)]
            bins = [jnp.minimum(
                (c * jnp.float32(N_BINS)).astype(jnp.int32), N_BINS - 1)
                for c in confs]
            diffs = [c - a for c, a in zip(confs, accs)]
            for k in range(unroll):
                plsc.addupdate_scatter(cnt_ref, [bins[k], lane], ones16)
                plsc.addupdate_scatter(sd_ref, [bins[k], lane], diffs[k])
            return 0

        lax.fori_loop(0, chunk // group, inner, 0)

    pltpu.sync_copy(cnt_ref, cnt_out.at[wid])
    pltpu.sync_copy(sd_ref, sd_out.at[wid])


def _final_body(cnt_ref, sd_ref, o_ref, *, n_total):
    cnt = jnp.sum(cnt_ref[...], axis=(0, 2))
    sd = jnp.sum(sd_ref[...], axis=(0, 2))
    denom = jnp.maximum(cnt, 1.0)
    contrib = jnp.where(cnt > 0.0,
                        (cnt / jnp.float32(n_total)) * jnp.abs(sd) / denom,
                        0.0)
    o_ref[0] = jnp.sum(contrib)


def kernel(accuracies, confidences):
    n = confidences.shape[0]
    n_per_w = n // _NW
    chunk = 8192
    if n_per_w % chunk != 0:
        chunk = n_per_w

    nbuf = 4
    mesh = plsc.VectorSubcoreMesh(core_axis_name="c", subcore_axis_name="s")
    buf_tree = [
        t for _ in range(nbuf)
        for t in (pltpu.VMEM((chunk,), jnp.float32),
                  pltpu.VMEM((chunk,), jnp.float32),
                  pltpu.SemaphoreType.DMA,
                  pltpu.SemaphoreType.DMA)
    ]
    sc_fn = pl.kernel(
        functools.partial(_sc_body, n_per_w=n_per_w, chunk=chunk, unroll=16,
                          nbuf=nbuf),
        mesh=mesh,
        compiler_params=pltpu.CompilerParams(needs_layout_passes=False),
        out_type=(
            jax.ShapeDtypeStruct((_NW, N_BINS, _L), jnp.float32),
            jax.ShapeDtypeStruct((_NW, N_BINS, _L), jnp.float32),
        ),
        scratch_types=[
            buf_tree,
            pltpu.VMEM((N_BINS, _L), jnp.float32),
            pltpu.VMEM((N_BINS, _L), jnp.float32),
        ],
    )
    cnt_parts, sd_parts = sc_fn(accuracies, confidences)

    out = pl.pallas_call(
        functools.partial(_final_body, n_total=n),
        out_shape=jax.ShapeDtypeStruct((1,), jnp.float32),
        out_specs=pl.BlockSpec(memory_space=pltpu.SMEM),
    )(cnt_parts, sd_parts)
    return out
